# baseline (speedup 1.0000x reference)
import functools

import jax
import jax.numpy as jnp
from jax import lax
from jax.experimental import pallas as pl
from jax.experimental.pallas import tpu as pltpu
from jax.experimental.pallas import tpu_sc as plsc

_C = 1000
_L = 16
_NS = 16
_NCORE = 2
_CH = 32
_RMAX = 63


def _row0(s):
    return (125 * s) // 2


def _score_body(x_ref, w_ref, e_ref, e_scr, *, kb):
    k = pl.program_id(1)
    xr = x_ref[0]
    t = jnp.sum(xr * w_ref[0][None, None, :], axis=2)
    s = jnp.sum(t, axis=1)
    e = jnp.exp(s)
    e_scr[k] = e.reshape(8, xr.shape[0] // 8)

    @pl.when(k == kb - 1)
    def _emit():
        e_ref[0] = e_scr[...]


def _scores(x, w128):
    B, N, F, H = x.shape
    nblk = 2000
    kb = N // nblk
    e4 = pl.pallas_call(
        functools.partial(_score_body, kb=kb),
        grid=(B, kb),
        in_specs=[pl.BlockSpec((1, nblk, F, H), lambda b_, k: (b_, k, 0, 0)),
                  pl.BlockSpec((1, H), lambda b_, k: (0, 0))],
        out_specs=pl.BlockSpec((1, kb, 8, nblk // 8), lambda b_, k: (b_, 0, 0, 0)),
        out_shape=jax.ShapeDtypeStruct((B, kb, 8, nblk // 8), jnp.float32),
        scratch_shapes=[pltpu.VMEM((kb, 8, nblk // 8), jnp.float32)],
    )(x, w128.reshape(1, H))
    return e4.reshape(B, 1, N)


def _sc_body(x_hbm, seg_hbm, cb_hbm, e_hbm, out_hbm,
             segall, ebufall, xbuf0, xbuf1, accbuf, outbuf,
             sem0, sem1, cbbuf, *, n_nodes, nf, nh, bpc):
    fh = nf * nh
    nhc = nh // _L
    core = lax.axis_index("c")
    s = lax.axis_index("s")
    c0 = _row0(s)
    nseg = _row0(s + 1) - c0

    pltpu.sync_copy(seg_hbm, segall.at[pl.ds(0, n_nodes)])
    pltpu.sync_copy(cb_hbm, cbbuf.at[pl.ds(0, _NS + 1)])

    n0 = cbbuf[pl.ds(s, _L)][0]
    n1 = cbbuf[pl.ds(s + 1, _L)][0]
    ck0 = n0 // _CH
    ck1 = (n1 + _CH - 1) // _CH
    nck = ck1 - ck0

    zero16 = jnp.zeros((_L,), jnp.float32)
    nj = fh // _L
    nacc = nj + 1

    for bl in range(bpc):
        b = core * bpc + bl

        pltpu.sync_copy(e_hbm.at[b, 0, :], ebufall.at[pl.ds(0, n_nodes)])

        def zero_row(r, carry):
            for j in range(nacc):
                accbuf[r, pl.ds(j * _L, _L)] = zero16
            return carry
        lax.fori_loop(0, _RMAX, zero_row, 0)

        def st_of(k):
            return jnp.minimum(k * _CH, n_nodes - _CH)

        def dma_start(k, xb, sem):
            st = st_of(k)
            pltpu.async_copy(x_hbm.at[b, pl.ds(st, _CH), :, :], xb, sem)

        def dma_wait(xb, sem):
            pltpu.make_async_copy(
                x_hbm.at[b, pl.ds(0, _CH), :, :], xb, sem).wait()

        def process(k, xb, carry):
            st = st_of(k)

            def node(r, cr):
                prev = cr[0]
                acc = cr[1:]
                g = st + r
                sg = segall[pl.ds(g, _L)][0]
                dd = g >= k * _CH
                inr = jnp.logical_and(
                    jnp.logical_and(sg >= c0, sg < c0 + nseg), dd)
                lc = jnp.where(dd, jnp.clip(sg - c0, 0, _RMAX - 1), prev)
                e16 = plsc.load_gather(
                    ebufall, [jnp.full((_L,), g, jnp.int32)])
                e16 = e16 * jnp.full((_L,), inr.astype(jnp.float32))
                contrib = tuple(
                    e16 * xb[r, f, pl.ds(c * _L, _L)]
                    for f in range(nf) for c in range(nhc)) + (e16,)

                def run_break():
                    pr = jnp.clip(prev, 0, _RMAX - 1)
                    for j in range(nacc):
                        accbuf[pr, pl.ds(j * _L, _L)] = acc[j]
                    return contrib

                def run_cont():
                    return tuple(a + cj for a, cj in zip(acc, contrib))

                newacc = lax.cond(lc != prev, run_break, run_cont)
                return (lc,) + newacc
            return lax.fori_loop(0, _CH, node, carry, unroll=2)

        @pl.when(nck > 0)
        def _prologue():
            dma_start(ck0, xbuf0, sem0)

        carry0 = (jnp.int32(-1),) + tuple(zero16 for _ in range(nacc))

        def pair(k2, cr):
            k = ck0 + 2 * k2

            def even(c):
                dma_wait(xbuf0, sem0)

                @pl.when(k + 1 < ck1)
                def _pf1():
                    dma_start(k + 1, xbuf1, sem1)
                return process(k, xbuf0, c)
            cr = lax.cond(k < ck1, even, lambda c: c, cr)

            def odd(c):
                dma_wait(xbuf1, sem1)

                @pl.when(k + 2 < ck1)
                def _pf2():
                    dma_start(k + 2, xbuf0, sem0)
                return process(k + 1, xbuf1, c)
            return lax.cond(k + 1 < ck1, odd, lambda c: c, cr)

        carry = lax.fori_loop(0, (nck + 1) // 2, pair, carry0)

        prf = jnp.clip(carry[0], 0, _RMAX - 1)
        for j in range(nacc):
            accbuf[prf, pl.ds(j * _L, _L)] = carry[1 + j]

        def fin_row(r, carry2):
            d16 = accbuf[r, pl.ds(fh, _L)]
            r16 = 1.0 / jnp.where(d16 > 0.0, d16, 1.0)
            for f in range(nf):
                for c in range(nhc):
                    outbuf[r, f, pl.ds(c * _L, _L)] = (
                        accbuf[r, pl.ds((f * nhc + c) * _L, _L)] * r16)
            return carry2
        lax.fori_loop(0, _RMAX, fin_row, 0)

        pltpu.sync_copy(outbuf.at[pl.ds(0, _RMAX - 1), :, :],
                        out_hbm.at[b, pl.ds(c0, _RMAX - 1), :, :])

        @pl.when(nseg == _RMAX)
        def _last_row():
            pltpu.sync_copy(outbuf.at[pl.ds(_RMAX - 1, 1), :, :],
                            out_hbm.at[b, pl.ds(c0 + _RMAX - 1, 1), :, :])


def kernel(x, hierarchy_mapping, W, b):
    B, N, F, H = x.shape
    FH = F * H
    w128 = (W[0] / F).astype(jnp.float32)
    seg = hierarchy_mapping.astype(jnp.int32)

    e3 = _scores(x, w128)

    bounds = jnp.array([_row0(s) for s in range(_NS + 1)], jnp.int32)
    cbounds = jnp.sum(seg[None, :] < bounds[:, None], axis=1).astype(jnp.int32)

    mesh = plsc.VectorSubcoreMesh(core_axis_name="c", subcore_axis_name="s",
                                  num_cores=_NCORE, num_subcores=_NS)
    bpc = B // _NCORE

    fn = functools.partial(
        pl.kernel,
        out_type=jax.ShapeDtypeStruct((B, _C, F, H), jnp.float32),
        mesh=mesh,
        scratch_types=[
            pltpu.VMEM((N + _L,), jnp.int32),
            pltpu.VMEM((N + _L,), jnp.float32),
            pltpu.VMEM((_CH, F, H), jnp.float32),
            pltpu.VMEM((_CH, F, H), jnp.float32),
            pltpu.VMEM((_RMAX, FH + _L), jnp.float32),
            pltpu.VMEM((_RMAX, F, H), jnp.float32),
            pltpu.SemaphoreType.DMA,
            pltpu.SemaphoreType.DMA,
            pltpu.VMEM((_NS + 1 + _L,), jnp.int32),
        ],
        compiler_params=pltpu.CompilerParams(use_tc_tiling_on_sc=True,
                                             needs_layout_passes=False),
    )(functools.partial(_sc_body, n_nodes=N, nf=F, nh=H, bpc=bpc))
    return fn(x, seg, cbounds, e3)


# speedup vs baseline: 1.0527x; 1.0527x over previous
import functools

import jax
import jax.numpy as jnp
from jax import lax
from jax.experimental import pallas as pl
from jax.experimental.pallas import tpu as pltpu
from jax.experimental.pallas import tpu_sc as plsc

_C = 1000
_L = 16
_NS = 16
_NCORE = 2
_CH = 32
_RMAX = 63


def _row0(s):
    return (125 * s) // 2


def _score_body(x_ref, w_ref, e_ref, e_scr, *, kb):
    k = pl.program_id(1)
    xr = x_ref[0]
    t = jnp.sum(xr * w_ref[0][None, None, :], axis=2)
    s = jnp.sum(t, axis=1)
    e = jnp.exp(s)
    e_scr[k] = e.reshape(8, xr.shape[0] // 8)

    @pl.when(k == kb - 1)
    def _emit():
        e_ref[0] = e_scr[...]


def _scores(x, w128):
    B, N, F, H = x.shape
    nblk = 2000
    kb = N // nblk
    e4 = pl.pallas_call(
        functools.partial(_score_body, kb=kb),
        grid=(B, kb),
        in_specs=[pl.BlockSpec((1, nblk, F, H), lambda b_, k: (b_, k, 0, 0)),
                  pl.BlockSpec((1, H), lambda b_, k: (0, 0))],
        out_specs=pl.BlockSpec((1, kb, 8, nblk // 8), lambda b_, k: (b_, 0, 0, 0)),
        out_shape=jax.ShapeDtypeStruct((B, kb, 8, nblk // 8), jnp.float32),
        scratch_shapes=[pltpu.VMEM((kb, 8, nblk // 8), jnp.float32)],
    )(x, w128.reshape(1, H))
    return e4.reshape(B, 1, N)


def _sc_body(x_hbm, seg_hbm, cb_hbm, e_hbm, out_hbm,
             segall, ebufall, xbuf0, xbuf1, accbuf, outbuf,
             sem0, sem1, cbbuf, *, n_nodes, nf, nh, bpc):
    fh = nf * nh
    nhc = nh // _L
    core = lax.axis_index("c")
    s = lax.axis_index("s")
    c0 = _row0(s)
    nseg = _row0(s + 1) - c0

    pltpu.sync_copy(seg_hbm, segall.at[pl.ds(0, n_nodes)])
    pltpu.sync_copy(cb_hbm, cbbuf.at[pl.ds(0, _NS + 1)])

    n0 = cbbuf[pl.ds(s, _L)][0]
    n1 = cbbuf[pl.ds(s + 1, _L)][0]
    ck0 = n0 // _CH
    ck1 = (n1 + _CH - 1) // _CH
    nck = ck1 - ck0

    zero16 = jnp.zeros((_L,), jnp.float32)
    nj = fh // _L
    nacc = nj + 1

    for bl in range(bpc):
        b = core * bpc + bl

        pltpu.sync_copy(e_hbm.at[b, 0, :], ebufall.at[pl.ds(0, n_nodes)])

        def zero_row(r, carry):
            for j in range(nacc):
                accbuf[r, pl.ds(j * _L, _L)] = zero16
            return carry
        lax.fori_loop(0, _RMAX, zero_row, 0)

        def st_of(k):
            return jnp.minimum(k * _CH, n_nodes - _CH)

        def dma_start(k, xb, sem):
            st = st_of(k)
            pltpu.async_copy(x_hbm.at[b, pl.ds(st, _CH), :, :], xb, sem)

        def dma_wait(xb, sem):
            pltpu.make_async_copy(
                x_hbm.at[b, pl.ds(0, _CH), :, :], xb, sem).wait()

        def process(k, xb, carry):
            st = st_of(k)

            def node(r, cr):
                prev = cr[0]
                acc = cr[1:]
                g = st + r
                sg = segall[pl.ds(g, _L)][0]
                dd = g >= k * _CH
                inr = jnp.logical_and(
                    jnp.logical_and(sg >= c0, sg < c0 + nseg), dd)
                lc = jnp.where(dd, jnp.clip(sg - c0, 0, _RMAX - 1), prev)
                e16 = plsc.load_gather(
                    ebufall, [jnp.full((_L,), g, jnp.int32)])
                e16 = e16 * jnp.full((_L,), inr.astype(jnp.float32))
                contrib = tuple(
                    e16 * xb[r, f, pl.ds(c * _L, _L)]
                    for f in range(nf) for c in range(nhc)) + (e16,)

                def run_break():
                    pr = jnp.clip(prev, 0, _RMAX - 1)
                    for j in range(nacc):
                        accbuf[pr, pl.ds(j * _L, _L)] = acc[j]
                    return contrib

                def run_cont():
                    return tuple(a + cj for a, cj in zip(acc, contrib))

                newacc = lax.cond(lc != prev, run_break, run_cont)
                return (lc,) + newacc
            return lax.fori_loop(0, _CH, node, carry)

        @pl.when(nck > 0)
        def _prologue():
            dma_start(ck0, xbuf0, sem0)

        carry0 = (jnp.int32(-1),) + tuple(zero16 for _ in range(nacc))

        def pair(k2, cr):
            k = ck0 + 2 * k2

            def even(c):
                dma_wait(xbuf0, sem0)

                @pl.when(k + 1 < ck1)
                def _pf1():
                    dma_start(k + 1, xbuf1, sem1)
                return process(k, xbuf0, c)
            cr = lax.cond(k < ck1, even, lambda c: c, cr)

            def odd(c):
                dma_wait(xbuf1, sem1)

                @pl.when(k + 2 < ck1)
                def _pf2():
                    dma_start(k + 2, xbuf0, sem0)
                return process(k + 1, xbuf1, c)
            return lax.cond(k + 1 < ck1, odd, lambda c: c, cr)

        carry = lax.fori_loop(0, (nck + 1) // 2, pair, carry0)

        prf = jnp.clip(carry[0], 0, _RMAX - 1)
        for j in range(nacc):
            accbuf[prf, pl.ds(j * _L, _L)] = carry[1 + j]

        def fin_row(r, carry2):
            d16 = accbuf[r, pl.ds(fh, _L)]
            r16 = 1.0 / jnp.where(d16 > 0.0, d16, 1.0)
            for f in range(nf):
                for c in range(nhc):
                    outbuf[r, f, pl.ds(c * _L, _L)] = (
                        accbuf[r, pl.ds((f * nhc + c) * _L, _L)] * r16)
            return carry2
        lax.fori_loop(0, _RMAX, fin_row, 0)

        pltpu.sync_copy(outbuf.at[pl.ds(0, _RMAX - 1), :, :],
                        out_hbm.at[b, pl.ds(c0, _RMAX - 1), :, :])

        @pl.when(nseg == _RMAX)
        def _last_row():
            pltpu.sync_copy(outbuf.at[pl.ds(_RMAX - 1, 1), :, :],
                            out_hbm.at[b, pl.ds(c0 + _RMAX - 1, 1), :, :])


def kernel(x, hierarchy_mapping, W, b):
    B, N, F, H = x.shape
    FH = F * H
    w128 = (W[0] / F).astype(jnp.float32)
    seg = hierarchy_mapping.astype(jnp.int32)

    e3 = _scores(x, w128)

    bounds = jnp.array([_row0(s) for s in range(_NS + 1)], jnp.int32)
    cbounds = jnp.sum(seg[None, :] < bounds[:, None], axis=1).astype(jnp.int32)

    mesh = plsc.VectorSubcoreMesh(core_axis_name="c", subcore_axis_name="s",
                                  num_cores=_NCORE, num_subcores=_NS)
    bpc = B // _NCORE

    fn = functools.partial(
        pl.kernel,
        out_type=jax.ShapeDtypeStruct((B, _C, F, H), jnp.float32),
        mesh=mesh,
        scratch_types=[
            pltpu.VMEM((N + _L,), jnp.int32),
            pltpu.VMEM((N + _L,), jnp.float32),
            pltpu.VMEM((_CH, F, H), jnp.float32),
            pltpu.VMEM((_CH, F, H), jnp.float32),
            pltpu.VMEM((_RMAX, FH + _L), jnp.float32),
            pltpu.VMEM((_RMAX, F, H), jnp.float32),
            pltpu.SemaphoreType.DMA,
            pltpu.SemaphoreType.DMA,
            pltpu.VMEM((_NS + 1 + _L,), jnp.int32),
        ],
        compiler_params=pltpu.CompilerParams(use_tc_tiling_on_sc=True,
                                             needs_layout_passes=False),
    )(functools.partial(_sc_body, n_nodes=N, nf=F, nh=H, bpc=bpc))
    return fn(x, seg, cbounds, e3)
